# baseline (speedup 1.0000x reference)
import functools

import jax
import jax.numpy as jnp
import numpy as np
from jax.experimental import pallas as pl
from jax.experimental.pallas import tpu as pltpu

X_DIM, E_DIM, G_DIM = 128, 16, 16
NUM_EMB, NUM_HEADS = 64, 2
N_NODES, N_EDGES, N_GRAPHS = 10000, 320000, 128
H = NUM_EMB
HH = NUM_HEADS
N_AUG = N_NODES + N_GRAPHS
E_AUG = N_EDGES + 2 * N_NODES + N_AUG


def _leaky(x):
    return jnp.where(x >= 0, x, 0.01 * x)


def _dot(a, b, hi=False):
    return jnp.dot(a, b, preferred_element_type=jnp.float32,
                   precision=jax.lax.Precision.HIGHEST if hi else None)


def _mlp3_body(x_ref, w1, b1, w2, b2, w3, b3, out_ref):
    h = _leaky(_dot(x_ref[...], w1[...]) + b1[...])
    h = _leaky(_dot(h, w2[...]) + b2[...])
    out_ref[...] = _dot(h, w3[...]) + b3[...]


def _mlp3(x, ws, bs, blk=None):
    n = x.shape[0]
    dout = ws[2].shape[1]
    if blk is None:
        blk = n
    grid = n // blk
    full = lambda s: pl.BlockSpec(s, lambda i: (0, 0))
    return pl.pallas_call(
        _mlp3_body,
        grid=(grid,),
        in_specs=[pl.BlockSpec((blk, x.shape[1]), lambda i: (i, 0))]
        + [full(w.shape) for pair in zip(ws, bs) for w in pair],
        out_specs=pl.BlockSpec((blk, dout), lambda i: (i, 0)),
        out_shape=jax.ShapeDtypeStruct((n, dout), jnp.float32),
    )(x, ws[0], bs[0], ws[1], bs[1], ws[2], bs[2])


def _ln_stats(o, onehot, onehot_t, inv_norm):
    rs = o.sum(-1, keepdims=True)
    rss = (o * o).sum(-1, keepdims=True)
    rs2 = jnp.concatenate([rs, rss], 1)
    stats = _dot(onehot_t, rs2, hi=True) * inv_norm
    per = _dot(onehot, stats, hi=True)
    m = per[:, 0:1]
    var = per[:, 1:2] - m * m
    return (o - m) * jax.lax.rsqrt(var + 1e-5)


def _ln_body(o_ref, oh_ref, oht_ref, invn_ref, out_ref):
    out_ref[...] = _ln_stats(o_ref[...], oh_ref[...], oht_ref[...], invn_ref[...])


def _ln_call(o, oh, oht, invn):
    return pl.pallas_call(
        _ln_body,
        out_shape=jax.ShapeDtypeStruct(o.shape, jnp.float32),
    )(o, oh, oht, invn)


def _genqkv_body(onorm_ref, aggp_ref, wg, bg, wq, bq, wk, bk, wv, bv, ws_, bs_,
                 bd_ref, ktab_ref, qqw_ref, vtab_ref, skip_ref):
    onorm = onorm_ref[...]
    agg = aggp_ref[0] + aggp_ref[1] + onorm
    gen = _dot(agg, wg[...]) + bg[...]
    xcat = jnp.concatenate([onorm, gen], 1)
    q = _dot(xcat, wq[...]) + bq[...]
    k = _dot(xcat, wk[...]) + bk[...]
    v = _dot(xcat, wv[...]) + bv[...]
    skip = _dot(xcat, ws_[...]) + bs_[...]
    qw = _dot(q, bd_ref[...], hi=True)
    ktab_ref[...] = k
    qqw_ref[...] = jnp.concatenate([q, qw], 1)
    vtab_ref[0] = v[:, :H]
    vtab_ref[1] = v[:, H:]
    skip_ref[...] = skip


def _genqkv_call(onorm, aggp, lw):
    n = onorm.shape[0]
    blk = 1688
    grid = n // blk
    row = lambda c: pl.BlockSpec((blk, c), lambda i: (i, 0))
    full = lambda s: pl.BlockSpec(s, lambda i: (0,) * len(s))
    return pl.pallas_call(
        _genqkv_body,
        grid=(grid,),
        in_specs=[row(H), pl.BlockSpec((2, blk, H), lambda i: (0, i, 0)),
                  full((H, H)), full((1, H)),
                  full((2 * H, 2 * H)), full((1, 2 * H)),
                  full((2 * H, 2 * H)), full((1, 2 * H)),
                  full((2 * H, 2 * H)), full((1, 2 * H)),
                  full((2 * H, 2 * H)), full((1, 2 * H)),
                  full((2 * H, 2 * H))],
        out_specs=(row(2 * H), row(4 * H),
                   pl.BlockSpec((2, blk, H), lambda i: (0, i, 0)), row(2 * H)),
        out_shape=(
            jax.ShapeDtypeStruct((n, 2 * H), jnp.float32),
            jax.ShapeDtypeStruct((n, 4 * H), jnp.float32),
            jax.ShapeDtypeStruct((2, n, H), jnp.float32),
            jax.ShapeDtypeStruct((n, 2 * H), jnp.float32),
        ),
    )(onorm, aggp, lw["gen_w"], lw["gen_b"], lw["q_w"], lw["q_b"],
      lw["k_w"], lw["k_b"], lw["v_w"], lw["v_b"], lw["skip_w"], lw["skip_b"],
      lw["bd_t"])


def _post1_body(o_ref, exv_ref, t_ref, den_ref, skip_ref, bdw_ref, dup_ref,
                wl, bl, cs_ref, oh_ref, o2_ref):
    num = exv_ref[...] + _dot(t_ref[...], bdw_ref[...], hi=True)
    denb = _dot(den_ref[...], dup_ref[...], hi=True)
    out = num / (denb + 1e-16) + skip_ref[...]
    l_h = _dot(out, wl[...]) + bl[...]
    cs_n = _dot(oh_ref[...], cs_ref[...], hi=True)
    o2_ref[...] = o_ref[...] + l_h * cs_n[:, :H] + cs_n[:, H:]


def _ffadd_body(o2_ref, ln2_ref, wf1, bf1, wf2, bf2, o_out_ref):
    f = _leaky(_dot(ln2_ref[...], wf1[...]) + bf1[...])
    f = _dot(f, wf2[...]) + bf2[...]
    o_out_ref[...] = o2_ref[...] + f


def _post_call(o, exv, t, den, skip, lw, oh, oht, invn):
    n = o.shape[0]
    blk = 1688
    grid = n // blk
    row = lambda c: pl.BlockSpec((blk, c), lambda i: (i, 0))
    full = lambda s: pl.BlockSpec(s, lambda i: (0,) * len(s))
    o2 = pl.pallas_call(
        _post1_body,
        grid=(grid,),
        in_specs=[row(H), row(2 * H), row(2 * H), row(2), row(2 * H),
                  full((2 * H, 2 * H)), full((2, 2 * H)),
                  full((2 * H, H)), full((1, H)),
                  full((N_GRAPHS, 2 * H)), row(N_GRAPHS)],
        out_specs=row(H),
        out_shape=jax.ShapeDtypeStruct((n, H), jnp.float32),
    )(o, exv, t, den, skip, lw["bd_w"], lw["dup"], lw["lin_w"], lw["lin_b"],
      lw["cs"], oh)
    ln2 = _ln_call(o2, oh, oht, invn)
    return pl.pallas_call(
        _ffadd_body,
        grid=(grid,),
        in_specs=[row(H), row(H), full((H, 4 * H)), full((1, 4 * H)),
                  full((4 * H, H)), full((1, H))],
        out_specs=row(H),
        out_shape=jax.ShapeDtypeStruct((n, H), jnp.float32),
    )(o2, ln2, lw["ff1_w"], lw["ff1_b"], lw["ff2_w"], lw["ff2_b"])


def _final_body(o_ref, ohn_t_ref, invcnt_ref, w1, b1, w2, b2, out_ref):
    pooled = _dot(ohn_t_ref[...], o_ref[:N_NODES], hi=True) * invcnt_ref[...]
    glob = jnp.concatenate([pooled, o_ref[N_NODES:]], 1)
    h = _leaky(_dot(glob, w1[...]) + b1[...])
    out_ref[...] = _dot(h, w2[...]) + b2[...]


def kernel(x, edge_attr, cond, edge_index, batch, params):
    n, g = N_NODES, N_GRAPHS
    n_aug = N_AUG
    p = params

    def wb(lin, width):
        w = lin["w"]
        b = lin.get("b")
        b = jnp.zeros((1, w.shape[1]), jnp.float32) if b is None else b.reshape(1, -1)
        return w, b

    x2h_w = [q["w"] for q in p["x2h"]]; x2h_b = [q["b"].reshape(1, -1) for q in p["x2h"]]
    e2h_w = [q["w"] for q in p["e2h"]]; e2h_b = [q["b"].reshape(1, -1) for q in p["e2h"]]
    c2h_w = [q["w"] for q in p["c2h"]]; c2h_b = [q["b"].reshape(1, -1) for q in p["c2h"]]
    fin_w = [q["w"] for q in p["final"]]; fin_b = [q["b"].reshape(1, -1) for q in p["final"]]

    dup = jnp.zeros((2, 2 * H), jnp.float32)
    dup = dup.at[0, :H].set(1.0).at[1, H:].set(1.0)

    layers = []
    for lp in p["layers"]:
        we = lp["trans"]["e"]["w"]
        bd_t = jnp.zeros((2 * H, 2 * H), jnp.float32)
        bd_t = bd_t.at[:H, :H].set(we[:, :H].T).at[H:, H:].set(we[:, H:].T)
        bd_w = jnp.zeros((2 * H, 2 * H), jnp.float32)
        bd_w = bd_w.at[:H, :H].set(we[:, :H]).at[H:, H:].set(we[:, H:])
        layers.append({
            "gen_w": lp["gen"]["w"], "gen_b": lp["gen"]["b"].reshape(1, -1),
            "q_w": lp["trans"]["q"]["w"], "q_b": lp["trans"]["q"]["b"].reshape(1, -1),
            "k_w": lp["trans"]["k"]["w"], "k_b": lp["trans"]["k"]["b"].reshape(1, -1),
            "v_w": lp["trans"]["v"]["w"], "v_b": lp["trans"]["v"]["b"].reshape(1, -1),
            "skip_w": lp["trans"]["skip"]["w"], "skip_b": lp["trans"]["skip"]["b"].reshape(1, -1),
            "bd_t": bd_t, "bd_w": bd_w, "dup": dup,
            "lin_w": lp["lin"]["w"], "lin_b": lp["lin"]["b"].reshape(1, -1),
            "ff1_w": lp["ff"][0]["w"], "ff1_b": lp["ff"][0]["b"].reshape(1, -1),
            "ff2_w": lp["ff"][1]["w"], "ff2_b": lp["ff"][1]["b"].reshape(1, -1),
            "cs_w": lp["cscale"]["w"], "cs_b": lp["cscale"]["b"].reshape(1, -1),
        })

    o = _mlp3(x, x2h_w, x2h_b, blk=2000)
    e = _mlp3(edge_attr, e2h_w, e2h_b, blk=8000)
    cc = _mlp3(cond, c2h_w, c2h_b)

    idt = edge_index.dtype
    u = jnp.arange(n, dtype=idt)
    vb = batch.astype(idt) + n
    src0 = jnp.concatenate([edge_index[0], u, vb])
    dst0 = jnp.concatenate([edge_index[1], vb, u])
    e_p = jnp.zeros((2 * n, H), jnp.float32).at[:, 0].set(1.0)
    aug_e0 = jnp.concatenate([e, e_p], 0)
    cnt = jax.ops.segment_sum(jnp.ones((dst0.shape[0],), jnp.float32), dst0,
                              num_segments=n_aug)
    loop_attr = jax.ops.segment_sum(aug_e0, dst0, num_segments=n_aug) \
        / jnp.clip(cnt, 1.0, None)[:, None]
    loops = jnp.arange(n_aug, dtype=idt)
    src = jnp.concatenate([src0, loops])
    dst = jnp.concatenate([dst0, loops])
    aug_e = jnp.concatenate([aug_e0, loop_attr], 0)
    aug_batch = jnp.concatenate([batch, jnp.arange(g, dtype=batch.dtype)])
    o = jnp.concatenate([o, cc], 0)

    onehot = (aug_batch[:, None] == jnp.arange(g)[None, :]).astype(jnp.float32)
    onehot_t = onehot.T
    inv_norm = (1.0 / (jnp.clip(onehot.sum(0), 1.0, None) * H)).reshape(1, g).T

    for lw, lp in zip(layers, p["layers"]):
        lw["cs"] = cc @ lw["cs_w"] + lw["cs_b"]

        o_norm = _ln_call(o, onehot, onehot_t, inv_norm)

        msg = jax.nn.relu(o_norm[src] + aug_e) + 1e-7
        agg1 = jax.ops.segment_sum(msg, dst, num_segments=n_aug)
        aggp = jnp.stack([agg1, jnp.zeros_like(agg1)])

        ktab, qqw, vtab, skip = _genqkv_call(o_norm, aggp, lw)

        q_d = qqw[dst, :2 * H].reshape(-1, HH, H)
        qw_d = qqw[dst, 2 * H:].reshape(-1, HH, H)
        k_s = ktab[src].reshape(-1, HH, H)
        s = (jnp.einsum('ehc,ehc->eh', q_d, k_s) +
             jnp.einsum('ec,ehc->eh', aug_e, qw_d)) * (1.0 / np.sqrt(H).astype(np.float32))
        amax = jax.ops.segment_max(s, dst, num_segments=n_aug)
        amax = jnp.where(jnp.isfinite(amax), amax, 0.0)

        ex = jnp.exp(s - amax[dst])
        den = jax.ops.segment_sum(ex, dst, num_segments=n_aug)
        v_s = jnp.stack([vtab[0][src], vtab[1][src]], 1)
        exv = jax.ops.segment_sum(ex[:, :, None] * v_s, dst, num_segments=n_aug)
        t = jax.ops.segment_sum(ex[:, :, None] * aug_e[:, None, :], dst,
                                num_segments=n_aug)

        o = _post_call(o, exv.reshape(n_aug, 2 * H), t.reshape(n_aug, 2 * H),
                       den, skip, lw, onehot, onehot_t, inv_norm)

    ohn_t = onehot_t[:, :n]
    invcnt = (1.0 / jnp.clip(onehot[:n].sum(0), 1.0, None)).reshape(g, 1)
    out = pl.pallas_call(
        _final_body,
        out_shape=jax.ShapeDtypeStruct((g, H), jnp.float32),
    )(o, ohn_t, invcnt, fin_w[0], fin_b[0], fin_w[1], fin_b[1])
    return out


# speedup vs baseline: 21.4596x; 21.4596x over previous
import functools

import jax
import jax.numpy as jnp
import numpy as np
from jax import lax
from jax.experimental import pallas as pl
from jax.experimental.pallas import tpu as pltpu, tpu_sc as plsc

X_DIM, E_DIM, G_DIM = 128, 16, 16
NUM_EMB, NUM_HEADS = 64, 2
N_NODES, N_EDGES, N_GRAPHS = 10000, 320000, 128
H = NUM_EMB
HH = NUM_HEADS
N_AUG = N_NODES + N_GRAPHS
E_AUG = N_EDGES + 2 * N_NODES + N_AUG


def _leaky(x):
    return jnp.where(x >= 0, x, 0.01 * x)


def _dot(a, b, hi=False):
    return jnp.dot(a, b, preferred_element_type=jnp.float32,
                   precision=jax.lax.Precision.HIGHEST if hi else None)


def _mlp3_body(x_ref, w1, b1, w2, b2, w3, b3, out_ref):
    h = _leaky(_dot(x_ref[...], w1[...]) + b1[...])
    h = _leaky(_dot(h, w2[...]) + b2[...])
    out_ref[...] = _dot(h, w3[...]) + b3[...]


def _mlp3(x, ws, bs, blk=None):
    n = x.shape[0]
    dout = ws[2].shape[1]
    if blk is None:
        blk = n
    grid = n // blk
    full = lambda s: pl.BlockSpec(s, lambda i: (0, 0))
    return pl.pallas_call(
        _mlp3_body,
        grid=(grid,),
        in_specs=[pl.BlockSpec((blk, x.shape[1]), lambda i: (i, 0))]
        + [full(w.shape) for pair in zip(ws, bs) for w in pair],
        out_specs=pl.BlockSpec((blk, dout), lambda i: (i, 0)),
        out_shape=jax.ShapeDtypeStruct((n, dout), jnp.float32),
    )(x, ws[0], bs[0], ws[1], bs[1], ws[2], bs[2])


def _ln_stats(o, onehot, onehot_t, inv_norm):
    rs = o.sum(-1, keepdims=True)
    rss = (o * o).sum(-1, keepdims=True)
    rs2 = jnp.concatenate([rs, rss], 1)
    stats = _dot(onehot_t, rs2, hi=True) * inv_norm
    per = _dot(onehot, stats, hi=True)
    m = per[:, 0:1]
    var = per[:, 1:2] - m * m
    return (o - m) * jax.lax.rsqrt(var + 1e-5)


def _ln_body(o_ref, oh_ref, oht_ref, invn_ref, out_ref):
    out_ref[...] = _ln_stats(o_ref[...], oh_ref[...], oht_ref[...], invn_ref[...])


def _ln_call(o, oh, oht, invn):
    return pl.pallas_call(
        _ln_body,
        out_shape=jax.ShapeDtypeStruct(o.shape, jnp.float32),
    )(o, oh, oht, invn)


def _genqkv_body(onorm_ref, aggp_ref, wg, bg, wq, bq, wk, bk, wv, bv, ws_, bs_,
                 bd_ref, ktab_ref, qqw_ref, vtab_ref, skip_ref):
    onorm = onorm_ref[...]
    agg = aggp_ref[0] + aggp_ref[1] + onorm
    gen = _dot(agg, wg[...]) + bg[...]
    xcat = jnp.concatenate([onorm, gen], 1)
    q = _dot(xcat, wq[...]) + bq[...]
    k = _dot(xcat, wk[...]) + bk[...]
    v = _dot(xcat, wv[...]) + bv[...]
    skip = _dot(xcat, ws_[...]) + bs_[...]
    qw = _dot(q, bd_ref[...], hi=True)
    ktab_ref[...] = k
    qqw_ref[...] = jnp.concatenate([q, qw], 1)
    vtab_ref[0] = v[:, :H]
    vtab_ref[1] = v[:, H:]
    skip_ref[...] = skip


def _genqkv_call(onorm, aggp, lw):
    n = onorm.shape[0]
    blk = 1688
    grid = n // blk
    row = lambda c: pl.BlockSpec((blk, c), lambda i: (i, 0))
    full = lambda s: pl.BlockSpec(s, lambda i: (0,) * len(s))
    return pl.pallas_call(
        _genqkv_body,
        grid=(grid,),
        in_specs=[row(H), pl.BlockSpec((2, blk, H), lambda i: (0, i, 0)),
                  full((H, H)), full((1, H)),
                  full((2 * H, 2 * H)), full((1, 2 * H)),
                  full((2 * H, 2 * H)), full((1, 2 * H)),
                  full((2 * H, 2 * H)), full((1, 2 * H)),
                  full((2 * H, 2 * H)), full((1, 2 * H)),
                  full((2 * H, 2 * H))],
        out_specs=(row(2 * H), row(4 * H),
                   pl.BlockSpec((2, blk, H), lambda i: (0, i, 0)), row(2 * H)),
        out_shape=(
            jax.ShapeDtypeStruct((n, 2 * H), jnp.float32),
            jax.ShapeDtypeStruct((n, 4 * H), jnp.float32),
            jax.ShapeDtypeStruct((2, n, H), jnp.float32),
            jax.ShapeDtypeStruct((n, 2 * H), jnp.float32),
        ),
    )(onorm, aggp, lw["gen_w"], lw["gen_b"], lw["q_w"], lw["q_b"],
      lw["k_w"], lw["k_b"], lw["v_w"], lw["v_b"], lw["skip_w"], lw["skip_b"],
      lw["bd_t"])


def _post1_body(o_ref, exv_ref, t_ref, den_ref, skip_ref, bdw_ref, dup_ref,
                wl, bl, cs_ref, oh_ref, o2_ref):
    num = exv_ref[...] + _dot(t_ref[...], bdw_ref[...], hi=True)
    denb = _dot(den_ref[...], dup_ref[...], hi=True)
    out = num / (denb + 1e-16) + skip_ref[...]
    l_h = _dot(out, wl[...]) + bl[...]
    cs_n = _dot(oh_ref[...], cs_ref[...], hi=True)
    o2_ref[...] = o_ref[...] + l_h * cs_n[:, :H] + cs_n[:, H:]


def _ffadd_body(o2_ref, ln2_ref, wf1, bf1, wf2, bf2, o_out_ref):
    f = _leaky(_dot(ln2_ref[...], wf1[...]) + bf1[...])
    f = _dot(f, wf2[...]) + bf2[...]
    o_out_ref[...] = o2_ref[...] + f


def _post_call(o, exv, t, den, skip, lw, oh, oht, invn):
    n = o.shape[0]
    blk = 1688
    grid = n // blk
    row = lambda c: pl.BlockSpec((blk, c), lambda i: (i, 0))
    full = lambda s: pl.BlockSpec(s, lambda i: (0,) * len(s))
    o2 = pl.pallas_call(
        _post1_body,
        grid=(grid,),
        in_specs=[row(H), row(2 * H), row(2 * H), row(2), row(2 * H),
                  full((2 * H, 2 * H)), full((2, 2 * H)),
                  full((2 * H, H)), full((1, H)),
                  full((N_GRAPHS, 2 * H)), row(N_GRAPHS)],
        out_specs=row(H),
        out_shape=jax.ShapeDtypeStruct((n, H), jnp.float32),
    )(o, exv, t, den, skip, lw["bd_w"], lw["dup"], lw["lin_w"], lw["lin_b"],
      lw["cs"], oh)
    ln2 = _ln_call(o2, oh, oht, invn)
    return pl.pallas_call(
        _ffadd_body,
        grid=(grid,),
        in_specs=[row(H), row(H), full((H, 4 * H)), full((1, 4 * H)),
                  full((4 * H, H)), full((1, H))],
        out_specs=row(H),
        out_shape=jax.ShapeDtypeStruct((n, H), jnp.float32),
    )(o2, ln2, lw["ff1_w"], lw["ff1_b"], lw["ff2_w"], lw["ff2_b"])


def _final_body(o_ref, ohn_t_ref, invcnt_ref, w1, b1, w2, b2, out_ref):
    pooled = _dot(ohn_t_ref[...], o_ref[:N_NODES], hi=True) * invcnt_ref[...]
    glob = jnp.concatenate([pooled, o_ref[N_NODES:]], 1)
    h = _leaky(_dot(glob, w1[...]) + b1[...])
    out_ref[...] = _dot(h, w2[...]) + b2[...]


C = 128
NPAD = 10240
EPAD = 352256
NCH_W = EPAD // (32 * C)
NCH_T = EPAD // (16 * C)
E2 = N_EDGES + 2 * N_NODES
E2PAD = 344064
NCH2 = E2PAD // (32 * C)

_MESH = plsc.VectorSubcoreMesh(core_axis_name="c", subcore_axis_name="s")


def _zero_rows(z_v, nrow, ncol):
    zero16 = jnp.zeros((16,), jnp.float32)
    nseg = ncol // 16

    def f(i, _):
        z_v[i // nseg, pl.ds((i % nseg) * 16, 16)] = zero16
        return 0
    lax.fori_loop(0, nrow * nseg, f, 0)


def _sc_loopattr(ee2, dstp):
    W = 80
    nd = NPAD // 16

    @functools.partial(
        pl.kernel,
        out_type=jax.ShapeDtypeStruct((2, NPAD, W), jnp.float32),
        mesh=_MESH,
        compiler_params=pltpu.CompilerParams(use_tc_tiling_on_sc=False, needs_layout_passes=False),
        scratch_types=[
            pltpu.VMEM((C,), jnp.int32),
            pltpu.VMEM((C, H), jnp.float32),
            pltpu.VMEM((C, W), jnp.float32),
            pltpu.VMEM((NPAD // 16, W), jnp.float32),
            pltpu.VMEM_SHARED((NPAD, W), jnp.float32),
        ],
    )
    def k(e_hbm, dst_hbm, out_hbm, dst_v, e_v, row_v, z_v, acc_sh):
        cid = lax.axis_index("c")
        sid = lax.axis_index("s")
        wid = sid * 2 + cid
        nt = NPAD // 16
        _zero_rows(z_v, nt, W)
        pltpu.sync_copy(z_v, acc_sh.at[pl.ds(sid * nt, nt)])
        plsc.subcore_barrier()
        one0 = jnp.where(lax.iota(jnp.int32, 16) == 0, 1.0, 0.0)

        def chunk(i, _):
            base = pl.multiple_of((wid * NCH2 + i) * C, C)
            pltpu.sync_copy(dst_hbm.at[pl.ds(base, C)], dst_v)
            pltpu.sync_copy(e_hbm.at[pl.ds(base, C)], e_v)

            def f(e, _):
                for j in range(H // 16):
                    row_v[e, pl.ds(16 * j, 16)] = e_v[e, pl.ds(16 * j, 16)]
                row_v[e, pl.ds(H, 16)] = one0
                return 0
            lax.fori_loop(0, C, f, 0, unroll=4)
            pltpu.sync_copy(row_v, acc_sh.at[dst_v], add=True)
            return 0
        lax.fori_loop(0, NCH2, chunk, 0)
        plsc.subcore_barrier()
        pltpu.sync_copy(acc_sh.at[pl.ds(sid * nd, nd)],
                        out_hbm.at[cid, pl.ds(sid * nd, nd)])

    return k(ee2, dstp)


def _sc_genconv(onorm_pad, aug_e, srcp, dstp):
    nd = NPAD // 16

    @functools.partial(
        pl.kernel,
        out_type=jax.ShapeDtypeStruct((2, NPAD, H), jnp.float32),
        mesh=_MESH,
        compiler_params=pltpu.CompilerParams(use_tc_tiling_on_sc=False, needs_layout_passes=False),
        scratch_types=[
            pltpu.VMEM((C,), jnp.int32),
            pltpu.VMEM((C,), jnp.int32),
            pltpu.VMEM((C, H), jnp.float32),
            pltpu.VMEM((C, H), jnp.float32),
            pltpu.VMEM((NPAD // 16, H), jnp.float32),
            pltpu.VMEM_SHARED((NPAD, H), jnp.float32),
            pltpu.SemaphoreType.DMA,
        ],
    )
    def k(tab_hbm, e_hbm, src_hbm, dst_hbm, out_hbm,
          src_v, dst_v, rows_v, msg_v, z_v, acc_sh, sem):
        cid = lax.axis_index("c")
        sid = lax.axis_index("s")
        wid = sid * 2 + cid
        nt = NPAD // 16
        _zero_rows(z_v, nt, H)
        pltpu.sync_copy(z_v, acc_sh.at[pl.ds(sid * nt, nt)])
        plsc.subcore_barrier()

        def chunk(i, _):
            base = pl.multiple_of((wid * NCH_W + i) * C, C)
            pltpu.sync_copy(src_hbm.at[pl.ds(base, C)], src_v)
            pltpu.sync_copy(dst_hbm.at[pl.ds(base, C)], dst_v)
            pltpu.async_copy(tab_hbm.at[src_v], rows_v, sem).wait()
            pltpu.sync_copy(e_hbm.at[pl.ds(base, C)], msg_v)
            nseg = H // 16

            def f(j, _):
                r = j // nseg
                col = (j % nseg) * 16
                msg_v[r, pl.ds(col, 16)] = jnp.maximum(
                    rows_v[r, pl.ds(col, 16)] + msg_v[r, pl.ds(col, 16)],
                    0.0) + 1e-7
                return 0
            lax.fori_loop(0, C * nseg, f, 0, unroll=8)
            pltpu.sync_copy(msg_v, acc_sh.at[dst_v], add=True)
            return 0
        lax.fori_loop(0, NCH_W, chunk, 0)
        plsc.subcore_barrier()
        pltpu.sync_copy(acc_sh.at[pl.ds(sid * nd, nd)],
                        out_hbm.at[cid, pl.ds(sid * nd, nd)])

    return k(onorm_pad, aug_e, srcp, dstp)


def _sc_passa(ktab_pad, qqw_pad, aug_e, srcp, dstp):
    @functools.partial(
        pl.kernel,
        out_type=(jax.ShapeDtypeStruct((2 * EPAD,), jnp.float32),
                  jax.ShapeDtypeStruct((32, 2 * NPAD), jnp.float32)),
        mesh=_MESH,
        compiler_params=pltpu.CompilerParams(use_tc_tiling_on_sc=False, needs_layout_passes=False),
        scratch_types=[
            pltpu.VMEM((C,), jnp.int32),
            pltpu.VMEM((C,), jnp.int32),
            pltpu.VMEM((C, 2 * H), jnp.float32),
            pltpu.VMEM((C, 4 * H), jnp.float32),
            pltpu.VMEM((C, H), jnp.float32),
            pltpu.VMEM((C,), jnp.float32),
            pltpu.VMEM((C,), jnp.float32),
            pltpu.VMEM((NPAD,), jnp.float32),
            pltpu.VMEM((NPAD,), jnp.float32),
            pltpu.VMEM((16,), jnp.int32),
            pltpu.VMEM((16,), jnp.float32),
            pltpu.SemaphoreType.DMA,
            pltpu.SemaphoreType.DMA,
            pltpu.SemaphoreType.DMA,
        ],
    )
    def k(k_hbm, qq_hbm, e_hbm, src_hbm, dst_hbm, s_hbm, amx_hbm,
          src_v, dst_v, k_v, qq_v, e_v, s0_v, s1_v, am0_v, am1_v,
          tmpk_v, tmps_v, sem, sem2, sem3):
        cid = lax.axis_index("c")
        sid = lax.axis_index("s")
        wid = sid * 2 + cid
        iota = lax.iota(jnp.int32, 16)
        neg = jnp.full((16,), -3.0e38, jnp.float32)
        nseg = NPAD // 16

        def init(i, _):
            am0_v[pl.ds(i * 16, 16)] = neg
            am1_v[pl.ds(i * 16, 16)] = neg
            return 0
        lax.fori_loop(0, nseg, init, 0)

        def segmax_update(am_ref, keys, sp):
            for d in (1, 2, 4, 8):
                idx = jnp.maximum(iota - d, 0)
                tmps_v[...] = sp
                kh = plsc.load_gather(tmpk_v, [idx])
                sh = plsc.load_gather(tmps_v, [idx])
                sp = jnp.where(kh == keys, jnp.maximum(sp, sh), sp)
            nxt = plsc.load_gather(tmpk_v, [jnp.minimum(iota + 1, 15)])
            last = (nxt != keys) | (iota == 15)
            cur = plsc.load_gather(am_ref, [keys])
            plsc.store_scatter(am_ref, [keys], jnp.maximum(cur, sp),
                               mask=last)

        def chunk(i, _):
            base = pl.multiple_of((wid * NCH_W + i) * C, C)
            pltpu.sync_copy(src_hbm.at[pl.ds(base, C)], src_v)
            pltpu.sync_copy(dst_hbm.at[pl.ds(base, C)], dst_v)
            cp1 = pltpu.async_copy(k_hbm.at[src_v], k_v, sem)
            cp2 = pltpu.async_copy(qq_hbm.at[dst_v], qq_v, sem2)
            cp3 = pltpu.async_copy(e_hbm.at[pl.ds(base, C)], e_v, sem3)
            cp1.wait()
            cp2.wait()
            cp3.wait()

            def group(gi, _):
                sacc0 = jnp.zeros((16,), jnp.float32)
                sacc1 = jnp.zeros((16,), jnp.float32)
                for l in range(16):
                    e = gi * 16 + l
                    a0 = k_v[e, pl.ds(0, 16)] * qq_v[e, pl.ds(0, 16)]
                    a1 = k_v[e, pl.ds(H, 16)] * qq_v[e, pl.ds(H, 16)]
                    for j in range(1, 4):
                        a0 = a0 + k_v[e, pl.ds(16 * j, 16)] * qq_v[e, pl.ds(16 * j, 16)]
                        a1 = a1 + k_v[e, pl.ds(H + 16 * j, 16)] * qq_v[e, pl.ds(H + 16 * j, 16)]
                    for j in range(4):
                        ev = e_v[e, pl.ds(16 * j, 16)]
                        a0 = a0 + ev * qq_v[e, pl.ds(2 * H + 16 * j, 16)]
                        a1 = a1 + ev * qq_v[e, pl.ds(3 * H + 16 * j, 16)]
                    m = iota == l
                    sacc0 = jnp.where(m, jnp.sum(a0), sacc0)
                    sacc1 = jnp.where(m, jnp.sum(a1), sacc1)
                s0vec = sacc0 * 0.125
                s1vec = sacc1 * 0.125
                s0_v[pl.ds(gi * 16, 16)] = s0vec
                s1_v[pl.ds(gi * 16, 16)] = s1vec
                dstvec = dst_v[pl.ds(gi * 16, 16)]
                keys, perm = plsc.sort_key_val(dstvec, iota)
                tmpk_v[...] = keys
                tmps_v[...] = s0vec
                s0p = plsc.load_gather(tmps_v, [perm])
                segmax_update(am0_v, keys, s0p)
                tmps_v[...] = s1vec
                s1p = plsc.load_gather(tmps_v, [perm])
                segmax_update(am1_v, keys, s1p)
                return 0
            lax.fori_loop(0, C // 16, group, 0)
            pltpu.sync_copy(s0_v, s_hbm.at[pl.ds(base, C)])
            pltpu.sync_copy(s1_v, s_hbm.at[pl.ds(EPAD + base, C)])
            return 0
        lax.fori_loop(0, NCH_W, chunk, 0)
        pltpu.sync_copy(am0_v, amx_hbm.at[wid, pl.ds(0, NPAD)])
        pltpu.sync_copy(am1_v, amx_hbm.at[wid, pl.ds(NPAD, NPAD)])

    return k(ktab_pad, qqw_pad, aug_e, srcp, dstp)


def _amax_reduce_body(in_ref, out_ref):
    out_ref[...] = jnp.max(in_ref[...], axis=0, keepdims=True)


def _la_body(p_ref, out_ref):
    s = p_ref[0, :N_AUG] + p_ref[1, :N_AUG]
    out_ref[...] = s[:, :H] / jnp.clip(s[:, H:H + 1], 1.0, None)


def _sc_passb1(vflat_pad, s2, amaxr, srcp, dstp):
    W = 80
    nd = NPAD // 16

    @functools.partial(
        pl.kernel,
        out_type=jax.ShapeDtypeStruct((2, NPAD, W), jnp.float32),
        mesh=_MESH,
        compiler_params=pltpu.CompilerParams(use_tc_tiling_on_sc=False, needs_layout_passes=False),
        scratch_types=[
            pltpu.VMEM((C,), jnp.int32),
            pltpu.VMEM((C,), jnp.int32),
            pltpu.VMEM((C,), jnp.int32),
            pltpu.VMEM((C,), jnp.float32),
            pltpu.VMEM((C, H), jnp.float32),
            pltpu.VMEM((C, W), jnp.float32),
            pltpu.VMEM((NPAD,), jnp.float32),
            pltpu.VMEM((320, W), jnp.float32),
            pltpu.VMEM((16,), jnp.float32),
            pltpu.VMEM_SHARED((NPAD, W), jnp.float32),
            pltpu.SemaphoreType.DMA,
        ],
    )
    def k(v_hbm, s_hbm, amx_hbm, src_hbm, dst_hbm, out_hbm,
          src_v, src2_v, dst_v, s_v, v_v, row_v, amx_v, z_v, exg_v,
          acc_sh, sem):
        cid = lax.axis_index("c")
        sid = lax.axis_index("s")
        _zero_rows(z_v, 320, W)
        pltpu.sync_copy(z_v, acc_sh.at[pl.ds(sid * 640, 320)])
        pltpu.sync_copy(z_v, acc_sh.at[pl.ds(sid * 640 + 320, 320)])
        pltpu.sync_copy(amx_hbm.at[pl.ds(cid * NPAD, NPAD)], amx_v)
        plsc.subcore_barrier()
        msk0 = jnp.where(lax.iota(jnp.int32, 16) == 0, 1.0, 0.0)

        def chunk(i, _):
            base = pl.multiple_of((sid * NCH_T + i) * C, C)
            pltpu.sync_copy(src_hbm.at[pl.ds(base, C)], src_v)
            pltpu.sync_copy(dst_hbm.at[pl.ds(base, C)], dst_v)
            pltpu.sync_copy(s_hbm.at[pl.ds(cid * EPAD + base, C)], s_v)

            def fidx(j, _):
                o = 16 * j
                src2_v[pl.ds(o, 16)] = src_v[pl.ds(o, 16)] + cid * NPAD
                return 0
            lax.fori_loop(0, C // 16, fidx, 0)
            pltpu.async_copy(v_hbm.at[src2_v], v_v, sem).wait()

            def fgroup(gi, _):
                o = 16 * gi
                am16 = plsc.load_gather(amx_v, [dst_v[pl.ds(o, 16)]])
                exvec = jnp.exp(s_v[pl.ds(o, 16)] - am16)
                for l in range(16):
                    e = o + l
                    b = jnp.full((16,), exvec[l], jnp.float32)
                    for j in range(4):
                        row_v[e, pl.ds(16 * j, 16)] = v_v[e, pl.ds(16 * j, 16)] * b
                    row_v[e, pl.ds(H, 16)] = b * msk0
                return 0
            lax.fori_loop(0, C // 16, fgroup, 0)
            pltpu.sync_copy(row_v, acc_sh.at[dst_v], add=True)
            return 0
        lax.fori_loop(0, NCH_T, chunk, 0)
        plsc.subcore_barrier()
        pltpu.sync_copy(acc_sh.at[pl.ds(sid * nd, nd)],
                        out_hbm.at[cid, pl.ds(sid * nd, nd)])

    return k(vflat_pad, s2, amaxr, srcp, dstp)


def _sc_passb2(aug_e, s2, amaxr, dstp):
    W = H
    nd = NPAD // 16

    @functools.partial(
        pl.kernel,
        out_type=jax.ShapeDtypeStruct((2, NPAD, W), jnp.float32),
        mesh=_MESH,
        compiler_params=pltpu.CompilerParams(use_tc_tiling_on_sc=False, needs_layout_passes=False),
        scratch_types=[
            pltpu.VMEM((C,), jnp.int32),
            pltpu.VMEM((C,), jnp.float32),
            pltpu.VMEM((C, H), jnp.float32),
            pltpu.VMEM((C, W), jnp.float32),
            pltpu.VMEM((NPAD,), jnp.float32),
            pltpu.VMEM((320, W), jnp.float32),
            pltpu.VMEM((16,), jnp.float32),
            pltpu.VMEM_SHARED((NPAD, W), jnp.float32),
        ],
    )
    def k(e_hbm, s_hbm, amx_hbm, dst_hbm, out_hbm,
          dst_v, s_v, e_v, row_v, amx_v, z_v, exg_v, acc_sh):
        cid = lax.axis_index("c")
        sid = lax.axis_index("s")
        _zero_rows(z_v, 320, W)
        pltpu.sync_copy(z_v, acc_sh.at[pl.ds(sid * 640, 320)])
        pltpu.sync_copy(z_v, acc_sh.at[pl.ds(sid * 640 + 320, 320)])
        pltpu.sync_copy(amx_hbm.at[pl.ds(cid * NPAD, NPAD)], amx_v)
        plsc.subcore_barrier()

        def chunk(i, _):
            base = pl.multiple_of((sid * NCH_T + i) * C, C)
            pltpu.sync_copy(dst_hbm.at[pl.ds(base, C)], dst_v)
            pltpu.sync_copy(s_hbm.at[pl.ds(cid * EPAD + base, C)], s_v)
            pltpu.sync_copy(e_hbm.at[pl.ds(base, C)], e_v)

            def fgroup(gi, _):
                o = 16 * gi
                am16 = plsc.load_gather(amx_v, [dst_v[pl.ds(o, 16)]])
                exvec = jnp.exp(s_v[pl.ds(o, 16)] - am16)
                for l in range(16):
                    e = o + l
                    b = jnp.full((16,), exvec[l], jnp.float32)
                    for j in range(4):
                        row_v[e, pl.ds(16 * j, 16)] = e_v[e, pl.ds(16 * j, 16)] * b
                return 0
            lax.fori_loop(0, C // 16, fgroup, 0)
            pltpu.sync_copy(row_v, acc_sh.at[dst_v], add=True)
            return 0
        lax.fori_loop(0, NCH_T, chunk, 0)
        plsc.subcore_barrier()
        pltpu.sync_copy(acc_sh.at[pl.ds(sid * nd, nd)],
                        out_hbm.at[cid, pl.ds(sid * nd, nd)])

    return k(aug_e, s2, amaxr, dstp)


def kernel(x, edge_attr, cond, edge_index, batch, params):
    n, g = N_NODES, N_GRAPHS
    n_aug = N_AUG
    p = params

    def wb(lin, width):
        w = lin["w"]
        b = lin.get("b")
        b = jnp.zeros((1, w.shape[1]), jnp.float32) if b is None else b.reshape(1, -1)
        return w, b

    x2h_w = [q["w"] for q in p["x2h"]]; x2h_b = [q["b"].reshape(1, -1) for q in p["x2h"]]
    e2h_w = [q["w"] for q in p["e2h"]]; e2h_b = [q["b"].reshape(1, -1) for q in p["e2h"]]
    c2h_w = [q["w"] for q in p["c2h"]]; c2h_b = [q["b"].reshape(1, -1) for q in p["c2h"]]
    fin_w = [q["w"] for q in p["final"]]; fin_b = [q["b"].reshape(1, -1) for q in p["final"]]

    dup = jnp.zeros((2, 2 * H), jnp.float32)
    dup = dup.at[0, :H].set(1.0).at[1, H:].set(1.0)

    layers = []
    for lp in p["layers"]:
        we = lp["trans"]["e"]["w"]
        bd_t = jnp.zeros((2 * H, 2 * H), jnp.float32)
        bd_t = bd_t.at[:H, :H].set(we[:, :H].T).at[H:, H:].set(we[:, H:].T)
        bd_w = jnp.zeros((2 * H, 2 * H), jnp.float32)
        bd_w = bd_w.at[:H, :H].set(we[:, :H]).at[H:, H:].set(we[:, H:])
        layers.append({
            "gen_w": lp["gen"]["w"], "gen_b": lp["gen"]["b"].reshape(1, -1),
            "q_w": lp["trans"]["q"]["w"], "q_b": lp["trans"]["q"]["b"].reshape(1, -1),
            "k_w": lp["trans"]["k"]["w"], "k_b": lp["trans"]["k"]["b"].reshape(1, -1),
            "v_w": lp["trans"]["v"]["w"], "v_b": lp["trans"]["v"]["b"].reshape(1, -1),
            "skip_w": lp["trans"]["skip"]["w"], "skip_b": lp["trans"]["skip"]["b"].reshape(1, -1),
            "bd_t": bd_t, "bd_w": bd_w, "dup": dup,
            "lin_w": lp["lin"]["w"], "lin_b": lp["lin"]["b"].reshape(1, -1),
            "ff1_w": lp["ff"][0]["w"], "ff1_b": lp["ff"][0]["b"].reshape(1, -1),
            "ff2_w": lp["ff"][1]["w"], "ff2_b": lp["ff"][1]["b"].reshape(1, -1),
            "cs_w": lp["cscale"]["w"], "cs_b": lp["cscale"]["b"].reshape(1, -1),
        })

    o = _mlp3(x, x2h_w, x2h_b, blk=2000)
    e = _mlp3(edge_attr, e2h_w, e2h_b, blk=8000)
    cc = _mlp3(cond, c2h_w, c2h_b)

    idt = edge_index.dtype
    u = jnp.arange(n, dtype=idt)
    vb = batch.astype(idt) + n
    src0 = jnp.concatenate([edge_index[0], u, vb])
    dst0 = jnp.concatenate([edge_index[1], vb, u])
    e_p = jnp.zeros((2 * n, H), jnp.float32).at[:, 0].set(1.0)
    aug_e0 = jnp.concatenate([e, e_p], 0)

    pad2 = E2PAD - E2
    dpad2 = (jnp.arange(pad2, dtype=jnp.int32) % 16) + n_aug
    dst0p = jnp.concatenate([dst0.astype(jnp.int32), dpad2])
    ee2p = jnp.concatenate([aug_e0, jnp.zeros((pad2, H), jnp.float32)], 0)
    la_part = _sc_loopattr(ee2p, dst0p)
    loop_attr = pl.pallas_call(
        _la_body, out_shape=jax.ShapeDtypeStruct((n_aug, H), jnp.float32),
    )(la_part)

    loops = jnp.arange(n_aug, dtype=idt)
    src = jnp.concatenate([src0, loops])
    dst = jnp.concatenate([dst0, loops])
    aug_e = jnp.concatenate([aug_e0, loop_attr], 0)
    aug_batch = jnp.concatenate([batch, jnp.arange(g, dtype=batch.dtype)])
    o = jnp.concatenate([o, cc], 0)

    pade = EPAD - E_AUG
    padidx = (jnp.arange(pade, dtype=jnp.int32) % 16) + n_aug
    srcp = jnp.concatenate([src.astype(jnp.int32), padidx])
    dstp = jnp.concatenate([dst.astype(jnp.int32), padidx])
    aug_e_pad = jnp.concatenate([aug_e, jnp.zeros((pade, H), jnp.float32)], 0)
    nodepad = ((0, NPAD - n_aug), (0, 0))

    onehot = (aug_batch[:, None] == jnp.arange(g)[None, :]).astype(jnp.float32)
    onehot_t = onehot.T
    inv_norm = (1.0 / (jnp.clip(onehot.sum(0), 1.0, None) * H)).reshape(1, g).T

    for lw, lp in zip(layers, p["layers"]):
        lw["cs"] = cc @ lw["cs_w"] + lw["cs_b"]

        o_norm = _ln_call(o, onehot, onehot_t, inv_norm)

        onorm_pad = jnp.pad(o_norm, nodepad)
        aggp = _sc_genconv(onorm_pad, aug_e_pad, srcp, dstp)[:, :n_aug]

        ktab, qqw, vtab, skip = _genqkv_call(o_norm, aggp, lw)

        s2, amax_part = _sc_passa(jnp.pad(ktab, nodepad), jnp.pad(qqw, nodepad),
                                  aug_e_pad, srcp, dstp)
        amaxr = pl.pallas_call(
            _amax_reduce_body,
            out_shape=jax.ShapeDtypeStruct((1, 2 * NPAD), jnp.float32),
        )(amax_part).reshape(2 * NPAD)

        vflat = jnp.pad(vtab, ((0, 0),) + nodepad).reshape(2 * NPAD, H)
        pb1 = _sc_passb1(vflat, s2, amaxr, srcp, dstp)[:, :n_aug]
        pb2 = _sc_passb2(aug_e_pad, s2, amaxr, dstp)[:, :n_aug]
        exv = jnp.concatenate([pb1[0, :, :H], pb1[1, :, :H]], 1)
        t = jnp.concatenate([pb2[0], pb2[1]], 1)
        den = jnp.stack([pb1[0, :, H], pb1[1, :, H]], 1)

        o = _post_call(o, exv, t, den, skip, lw, onehot, onehot_t, inv_norm)

    ohn_t = onehot_t[:, :n]
    invcnt = (1.0 / jnp.clip(onehot[:n].sum(0), 1.0, None)).reshape(g, 1)
    out = pl.pallas_call(
        _final_body,
        out_shape=jax.ShapeDtypeStruct((g, H), jnp.float32),
    )(o, ohn_t, invcnt, fin_w[0], fin_b[0], fin_w[1], fin_b[1])
    return out
